# Initial kernel scaffold; baseline (speedup 1.0000x reference)
import jax
import jax.numpy as jnp
from jax.experimental import pallas as pl


def kernel(occurrence_count, costs, valid, segment_ids):
    raise NotImplementedError("write your pallas kernel here")


# speedup vs baseline: 4.1872x; 4.1872x over previous
import jax
import jax.numpy as jnp
from jax.experimental import pallas as pl


P = 16
S = 512


def _tc_body(occ_ref, costs_ref, valid_ref, seg_ref, out_ref):
    x = occ_ref[...]
    c = costs_ref[...]
    m = jax.lax.dot_general(
        x, c, (((1,), (1,)), ((), ())),
        preferred_element_type=jnp.float32)
    seg = seg_ref[0, 0, :]
    bt = x.shape[0]
    pid = jax.lax.broadcasted_iota(jnp.int32, (bt, P), 1)
    onehot = seg[:, None] == pid
    s = jnp.sum(jnp.where(onehot, m, 0.0), axis=1)
    vf = valid_ref[0, :]
    vq = jnp.sum(jnp.where(onehot, jnp.broadcast_to(vf[None, :], (bt, P)), 0.0),
                 axis=1) > 0.5
    out = jnp.where(vq, jnp.log(jnp.where(vq, s, 1.0)), 0.0)
    out_ref[0, 0, :] = out


def kernel(occurrence_count, costs, valid, segment_ids):
    T = occurrence_count.shape[0]
    BT = 1024
    nb = T // BT
    seg3 = segment_ids.reshape(nb, 1, BT)
    valid_f = valid.astype(jnp.float32).reshape(1, P)
    out = pl.pallas_call(
        _tc_body,
        grid=(nb,),
        in_specs=[
            pl.BlockSpec((BT, S), lambda i: (i, 0)),
            pl.BlockSpec((P, S), lambda i: (0, 0)),
            pl.BlockSpec((1, P), lambda i: (0, 0)),
            pl.BlockSpec((1, 1, BT), lambda i: (i, 0, 0)),
        ],
        out_specs=pl.BlockSpec((1, 1, BT), lambda i: (i, 0, 0)),
        out_shape=jax.ShapeDtypeStruct((nb, 1, BT), jnp.float32),
    )(occurrence_count, costs, valid_f, seg3)
    return out.reshape(T)
